# baseline (speedup 1.0000x reference)
import functools

import jax
import jax.numpy as jnp
from jax import lax
from jax.experimental import pallas as pl
from jax.experimental.pallas import tpu as pltpu
from jax.experimental.pallas import tpu_sc as plsc

_N = 10000
_E = 320000
_D = 128
_C = 40

_NC = 2
_NS = 16
_NW = _NC * _NS
_EW = _E // _NW
_CH = 100
_SB = 20
_NSUP = _EW // (_SB * _CH)
_SLAB = 624
_ZB = 16
_RD = 16

_BLK = 2000


def _seg_body(m_hbm, src_hbm, dst_hbm, out_hbm,
              srcA, srcB, dstA, dstB, r0, r1, zbuf_v, acc_sh,
              g0, g1, s0, s1, zsem):
    srcs = (srcA, srcB)
    dsts = (dstA, dstB)
    rows = (r0, r1)
    gsem = (g0, g1)
    ssem = (s0, s1)
    cid = lax.axis_index("c")
    sid = lax.axis_index("s")
    wid = cid * _NS + sid

    zero16 = jnp.zeros((16,), jnp.float32)

    def _zrow(i, carry):
        for j in range(_D // 16):
            zbuf_v[i, pl.ds(j * 16, 16)] = zero16
        return carry

    lax.fori_loop(0, _ZB, _zrow, 0)

    base = sid * _SLAB
    nz = jnp.where(sid == _NS - 1, 640 // _ZB, _SLAB // _ZB)

    def _ziss(r, carry):
        pltpu.async_copy(zbuf_v, acc_sh.at[pl.ds(base + r * _ZB, _ZB)], zsem)
        return carry

    lax.fori_loop(0, nz, _ziss, 0)

    pltpu.sync_copy(src_hbm.at[wid, 0], srcA)
    pltpu.sync_copy(dst_hbm.at[wid, 0], dstA)

    def _g(j, b, sv):
        pltpu.async_copy(m_hbm.at[sv.at[j]], rows[b], gsem[b])

    def _gwait(b):
        pltpu.make_async_copy(m_hbm.at[srcA.at[0]], rows[b], gsem[b]).wait()

    def _s(j, b, dv):
        pltpu.async_copy(rows[b], acc_sh.at[dv.at[j]], ssem[b], add=True)

    def _swait(b):
        pltpu.make_async_copy(rows[b], acc_sh.at[dstA.at[0]], ssem[b]).wait()

    _g(0, 0, srcA)

    def _zdrain(r, carry):
        pltpu.make_async_copy(zbuf_v, acc_sh.at[pl.ds(base, _ZB)], zsem).wait()
        return carry

    lax.fori_loop(0, nz, _zdrain, 0)

    plsc.subcore_barrier()

    for sc in range(_NSUP):
        p = sc % 2
        sv = srcs[p]
        dv = dsts[p]
        first = sc == 0
        last = sc == _NSUP - 1

        def _rev(t, carry):
            for b in range(2):
                j = 2 * t + b
                _gwait(b)
                _s(j, b, dv)
                if first:
                    @pl.when(j >= 1)
                    def _():
                        _swait(1 - b)
                else:
                    _swait(1 - b)
                _g(j + 1, 1 - b, sv)
            return carry

        lax.fori_loop(0, (_SB - 2) // 2, _rev, 0)

        if not last:
            pltpu.sync_copy(src_hbm.at[wid, sc + 1], srcs[1 - p])
            pltpu.sync_copy(dst_hbm.at[wid, sc + 1], dsts[1 - p])

        _gwait(0)
        _s(_SB - 2, 0, dv)
        _swait(1)
        _g(_SB - 1, 1, sv)
        _gwait(1)
        _s(_SB - 1, 1, dv)
        _swait(0)
        if not last:
            _g(0, 0, srcs[1 - p])
        else:
            _swait(1)

    plsc.subcore_barrier()

    nrd = jnp.where(sid == _NS - 1, 640 // _RD, _SLAB // _RD)

    def _wiss(r, carry):
        off = base + r * _RD
        pltpu.async_copy(acc_sh.at[pl.ds(off, _RD)],
                         out_hbm.at[cid, pl.ds(off, _RD)], zsem)
        return carry

    lax.fori_loop(0, nrd, _wiss, 0)

    def _wdrain(r, carry):
        pltpu.make_async_copy(acc_sh.at[pl.ds(base, _RD)],
                              out_hbm.at[cid, pl.ds(base, _RD)], zsem).wait()
        return carry

    lax.fori_loop(0, nrd, _wdrain, 0)


@functools.cache
def _seg_sum_kernel():
    return functools.partial(
        pl.kernel,
        out_type=jax.ShapeDtypeStruct((_NC, _N, _D), jnp.float32),
        mesh=plsc.VectorSubcoreMesh(core_axis_name="c", subcore_axis_name="s",
                                    num_cores=_NC, num_subcores=_NS),
        scratch_types=[
            pltpu.VMEM((_SB, _CH), jnp.int32),
            pltpu.VMEM((_SB, _CH), jnp.int32),
            pltpu.VMEM((_SB, _CH), jnp.int32),
            pltpu.VMEM((_SB, _CH), jnp.int32),
            pltpu.VMEM((_CH, _D), jnp.float32),
            pltpu.VMEM((_CH, _D), jnp.float32),
            pltpu.VMEM((_ZB, _D), jnp.float32),
            pltpu.VMEM_SHARED((_N, _D), jnp.float32),
            pltpu.SemaphoreType.DMA,
            pltpu.SemaphoreType.DMA,
            pltpu.SemaphoreType.DMA,
            pltpu.SemaphoreType.DMA,
            pltpu.SemaphoreType.DMA,
        ],
    )(_seg_body)


def _seg_sum(m, src, dst):
    return _seg_sum_kernel()(m, src, dst)


def _enc_body(x_ref, w1_ref, b1_ref, w2_ref, o_ref):
    h = jnp.dot(x_ref[...], w1_ref[...],
                preferred_element_type=jnp.float32) + b1_ref[...]
    o_ref[...] = jnp.dot(h, w2_ref[...], preferred_element_type=jnp.float32)


def _gcn_body(p_ref, b_ref, w_ref, o_ref):
    h = jax.nn.relu(p_ref[0] + p_ref[1] + b_ref[...])
    o_ref[...] = jnp.dot(h, w_ref[...], preferred_element_type=jnp.float32)


def _dec_body(p_ref, b_ref, wd_ref, bd_ref, o_ref):
    h = jax.nn.relu(p_ref[0] + p_ref[1] + b_ref[...])
    o = jnp.dot(h, wd_ref[...], preferred_element_type=jnp.float32) + bd_ref[...]
    m = jnp.max(o, axis=-1, keepdims=True)
    s = o - m
    lse = jnp.log(jnp.sum(jnp.exp(s), axis=-1, keepdims=True))
    o_ref[...] = s - lse


_enc_call = pl.pallas_call(
    _enc_body,
    grid=(_N // _BLK,),
    in_specs=[
        pl.BlockSpec((_BLK, _D), lambda i: (i, 0)),
        pl.BlockSpec((_D, _D), lambda i: (0, 0)),
        pl.BlockSpec((1, _D), lambda i: (0, 0)),
        pl.BlockSpec((_D, _D), lambda i: (0, 0)),
    ],
    out_specs=pl.BlockSpec((_BLK, _D), lambda i: (i, 0)),
    out_shape=jax.ShapeDtypeStruct((_N, _D), jnp.float32),
)

_gcn_call = pl.pallas_call(
    _gcn_body,
    grid=(_N // _BLK,),
    in_specs=[
        pl.BlockSpec((_NC, _BLK, _D), lambda i: (0, i, 0)),
        pl.BlockSpec((1, _D), lambda i: (0, 0)),
        pl.BlockSpec((_D, _D), lambda i: (0, 0)),
    ],
    out_specs=pl.BlockSpec((_BLK, _D), lambda i: (i, 0)),
    out_shape=jax.ShapeDtypeStruct((_N, _D), jnp.float32),
)

_dec_call = pl.pallas_call(
    _dec_body,
    grid=(_N // _BLK,),
    in_specs=[
        pl.BlockSpec((_NC, _BLK, _D), lambda i: (0, i, 0)),
        pl.BlockSpec((1, _D), lambda i: (0, 0)),
        pl.BlockSpec((_D, _C), lambda i: (0, 0)),
        pl.BlockSpec((1, _C), lambda i: (0, 0)),
    ],
    out_specs=pl.BlockSpec((_BLK, _C), lambda i: (i, 0)),
    out_shape=jax.ShapeDtypeStruct((_N, _C), jnp.float32),
)


def kernel(x, edge_index, enc_W, enc_b, gcn_W0, gcn_b0, gcn_W1, gcn_b1,
           dec_W, dec_b):
    src = edge_index[0].reshape(_NW, _NSUP, _SB, _CH)
    dst = edge_index[1].reshape(_NW, _NSUP, _SB, _CH)
    m0 = _enc_call(x, enc_W, enc_b.reshape(1, _D), gcn_W0)
    p0 = _seg_sum(m0, src, dst)
    m1 = _gcn_call(p0, gcn_b0.reshape(1, _D), gcn_W1)
    p1 = _seg_sum(m1, src, dst)
    return _dec_call(p1, gcn_b1.reshape(1, _D), dec_W, dec_b.reshape(1, _C))


# speedup vs baseline: 1.2472x; 1.2472x over previous
import functools

import jax
import jax.numpy as jnp
from jax import lax
from jax.experimental import pallas as pl
from jax.experimental.pallas import tpu as pltpu
from jax.experimental.pallas import tpu_sc as plsc

_N = 10000
_E = 320000
_D = 128
_C = 40

_NC = 2
_NS = 16
_NW = _NC * _NS
_EW = _E // _NW
_CH = 100
_SB = 25
_NSUP = _EW // (_SB * _CH)
_NB = 3
_SLAB = 624
_ZB = 16
_RD = 16

_BLK = 2000


def _seg_body(m_hbm, src_hbm, dst_hbm, out_hbm,
              src_v, dst_v, r0, r1, r2, zbuf_v, acc_sh,
              g0, g1, g2, s0, s1, s2, zsem):
    rows = (r0, r1, r2)
    gsem = (g0, g1, g2)
    ssem = (s0, s1, s2)
    cid = lax.axis_index("c")
    sid = lax.axis_index("s")
    wid = cid * _NS + sid

    zero16 = jnp.zeros((16,), jnp.float32)

    def _zrow(i, carry):
        for j in range(_D // 16):
            zbuf_v[i, pl.ds(j * 16, 16)] = zero16
        return carry

    lax.fori_loop(0, _ZB, _zrow, 0)

    base = sid * _SLAB
    nz = jnp.where(sid == _NS - 1, 640 // _ZB, _SLAB // _ZB)

    def _ziss(r, carry):
        pltpu.async_copy(zbuf_v, acc_sh.at[pl.ds(base + r * _ZB, _ZB)], zsem)
        return carry

    lax.fori_loop(0, nz, _ziss, 0)

    pltpu.sync_copy(src_hbm.at[wid, 0], src_v)
    pltpu.sync_copy(dst_hbm.at[wid, 0], dst_v)

    def _g(j, b):
        pltpu.async_copy(m_hbm.at[src_v.at[j]], rows[b], gsem[b])

    def _gwait(b):
        pltpu.make_async_copy(m_hbm.at[src_v.at[0]], rows[b], gsem[b]).wait()

    def _s(j, b):
        pltpu.async_copy(rows[b], acc_sh.at[dst_v.at[j]], ssem[b], add=True)

    def _swait(b):
        pltpu.make_async_copy(rows[b], acc_sh.at[dst_v.at[0]], ssem[b]).wait()

    _g(0, 0)
    _g(1, 1)

    def _zdrain(r, carry):
        pltpu.make_async_copy(zbuf_v, acc_sh.at[pl.ds(base, _ZB)], zsem).wait()
        return carry

    lax.fori_loop(0, nz, _zdrain, 0)

    plsc.subcore_barrier()

    for sc in range(_NSUP):
        if sc:
            pltpu.sync_copy(src_hbm.at[wid, sc], src_v)
            pltpu.sync_copy(dst_hbm.at[wid, sc], dst_v)

            _g(0, 0)
            _g(1, 1)

        def _rev(t, carry):
            for b in range(3):
                j = 3 * t + b
                _gwait(b)
                _s(j, b)
                bn = (b + 2) % 3

                @pl.when(j >= 1)
                def _():
                    _swait(bn)

                @pl.when(j + 2 < _SB)
                def _():
                    _g(j + 2, bn)
            return carry

        lax.fori_loop(0, _SB // 3, _rev, 0)

        _gwait(0)
        _s(_SB - 1, 0)
        _swait(2)
        _swait(0)

    plsc.subcore_barrier()

    nrd = jnp.where(sid == _NS - 1, 640 // _RD, _SLAB // _RD)

    def _wiss(r, carry):
        off = base + r * _RD
        pltpu.async_copy(acc_sh.at[pl.ds(off, _RD)],
                         out_hbm.at[cid, pl.ds(off, _RD)], zsem)
        return carry

    lax.fori_loop(0, nrd, _wiss, 0)

    def _wdrain(r, carry):
        pltpu.make_async_copy(acc_sh.at[pl.ds(base, _RD)],
                              out_hbm.at[cid, pl.ds(base, _RD)], zsem).wait()
        return carry

    lax.fori_loop(0, nrd, _wdrain, 0)


@functools.cache
def _seg_sum_kernel():
    return functools.partial(
        pl.kernel,
        out_type=jax.ShapeDtypeStruct((_NC, _N, _D), jnp.float32),
        mesh=plsc.VectorSubcoreMesh(core_axis_name="c", subcore_axis_name="s",
                                    num_cores=_NC, num_subcores=_NS),
        scratch_types=[
            pltpu.VMEM((_SB, _CH), jnp.int32),
            pltpu.VMEM((_SB, _CH), jnp.int32),
            pltpu.VMEM((_CH, _D), jnp.float32),
            pltpu.VMEM((_CH, _D), jnp.float32),
            pltpu.VMEM((_CH, _D), jnp.float32),
            pltpu.VMEM((_ZB, _D), jnp.float32),
            pltpu.VMEM_SHARED((_N, _D), jnp.float32),
            pltpu.SemaphoreType.DMA,
            pltpu.SemaphoreType.DMA,
            pltpu.SemaphoreType.DMA,
            pltpu.SemaphoreType.DMA,
            pltpu.SemaphoreType.DMA,
            pltpu.SemaphoreType.DMA,
            pltpu.SemaphoreType.DMA,
        ],
    )(_seg_body)


def _seg_sum(m, src, dst):
    return _seg_sum_kernel()(m, src, dst)


def _enc_body(x_ref, w1_ref, b1_ref, w2_ref, o_ref):
    h = jnp.dot(x_ref[...], w1_ref[...],
                preferred_element_type=jnp.float32) + b1_ref[...]
    o_ref[...] = jnp.dot(h, w2_ref[...], preferred_element_type=jnp.float32)


def _gcn_body(p_ref, b_ref, w_ref, o_ref):
    h = jax.nn.relu(p_ref[0] + p_ref[1] + b_ref[...])
    o_ref[...] = jnp.dot(h, w_ref[...], preferred_element_type=jnp.float32)


def _dec_body(p_ref, b_ref, wd_ref, bd_ref, o_ref):
    h = jax.nn.relu(p_ref[0] + p_ref[1] + b_ref[...])
    o = jnp.dot(h, wd_ref[...], preferred_element_type=jnp.float32) + bd_ref[...]
    m = jnp.max(o, axis=-1, keepdims=True)
    s = o - m
    lse = jnp.log(jnp.sum(jnp.exp(s), axis=-1, keepdims=True))
    o_ref[...] = s - lse


_enc_call = pl.pallas_call(
    _enc_body,
    grid=(_N // _BLK,),
    in_specs=[
        pl.BlockSpec((_BLK, _D), lambda i: (i, 0)),
        pl.BlockSpec((_D, _D), lambda i: (0, 0)),
        pl.BlockSpec((1, _D), lambda i: (0, 0)),
        pl.BlockSpec((_D, _D), lambda i: (0, 0)),
    ],
    out_specs=pl.BlockSpec((_BLK, _D), lambda i: (i, 0)),
    out_shape=jax.ShapeDtypeStruct((_N, _D), jnp.float32),
)

_gcn_call = pl.pallas_call(
    _gcn_body,
    grid=(_N // _BLK,),
    in_specs=[
        pl.BlockSpec((_NC, _BLK, _D), lambda i: (0, i, 0)),
        pl.BlockSpec((1, _D), lambda i: (0, 0)),
        pl.BlockSpec((_D, _D), lambda i: (0, 0)),
    ],
    out_specs=pl.BlockSpec((_BLK, _D), lambda i: (i, 0)),
    out_shape=jax.ShapeDtypeStruct((_N, _D), jnp.float32),
)

_dec_call = pl.pallas_call(
    _dec_body,
    grid=(_N // _BLK,),
    in_specs=[
        pl.BlockSpec((_NC, _BLK, _D), lambda i: (0, i, 0)),
        pl.BlockSpec((1, _D), lambda i: (0, 0)),
        pl.BlockSpec((_D, _C), lambda i: (0, 0)),
        pl.BlockSpec((1, _C), lambda i: (0, 0)),
    ],
    out_specs=pl.BlockSpec((_BLK, _C), lambda i: (i, 0)),
    out_shape=jax.ShapeDtypeStruct((_N, _C), jnp.float32),
)


def kernel(x, edge_index, enc_W, enc_b, gcn_W0, gcn_b0, gcn_W1, gcn_b1,
           dec_W, dec_b):
    src = edge_index[0].reshape(_NW, _NSUP, _SB, _CH)
    dst = edge_index[1].reshape(_NW, _NSUP, _SB, _CH)
    m0 = _enc_call(x, enc_W, enc_b.reshape(1, _D), gcn_W0)
    p0 = _seg_sum(m0, src, dst)
    m1 = _gcn_call(p0, gcn_b0.reshape(1, _D), gcn_W1)
    p1 = _seg_sum(m1, src, dst)
    return _dec_call(p1, gcn_b1.reshape(1, _D), dec_W, dec_b.reshape(1, _C))
